# Initial kernel scaffold; baseline (speedup 1.0000x reference)
import jax
import jax.numpy as jnp
from jax.experimental import pallas as pl


def kernel(x, gate_w, Wg, Wu, Wd, Sg, Su, Sd):
    raise NotImplementedError("write your pallas kernel here")


# speedup vs baseline: 1.8107x; 1.8107x over previous
import functools

import jax
import jax.numpy as jnp
from jax import lax
from jax.experimental import pallas as pl
from jax.experimental.pallas import tpu as pltpu
from jax.experimental.pallas import tpu_sc as plsc

T = 2048
H = 1024
E = 8
KTOP = 2
IDIM = 2752
BLK = 512
IC = 688
NIC = IDIM // IC
NBR = 15
PADR = NBR * BLK

_NW = 32
_SLOTS = KTOP * T
_SPW = _SLOTS // _NW
_SUB = 32
_NSUB = _SPW // _SUB


def _router_body(x_ref, gw_ref, p_ref, w_ref, meta_ref):
    x = x_ref[...]
    gw = gw_ref[...]
    logits = lax.dot_general(x, gw, (((1,), (1,)), ((), ())),
                             preferred_element_type=jnp.float32)
    scores = jax.nn.softmax(logits, axis=-1)

    eidx = lax.broadcasted_iota(jnp.int32, (T, E), 1)
    m1 = jnp.max(scores, axis=-1, keepdims=True)
    a1 = jnp.min(jnp.where(scores == m1, eidx, E), axis=-1, keepdims=True)
    s2 = jnp.where(eidx == a1, -jnp.inf, scores)
    m2 = jnp.max(s2, axis=-1, keepdims=True)
    a2 = jnp.min(jnp.where(s2 == m2, eidx, E), axis=-1, keepdims=True)
    denom = m1 + m2 + 1e-20
    w1 = m1 / denom
    w2 = m2 / denom

    oh1 = (eidx == a1).astype(jnp.float32)
    oh2 = (eidx == a2).astype(jnp.float32)
    ofull = jnp.concatenate([oh1, oh2], axis=0)

    cnt = jnp.sum(ofull, axis=0, keepdims=True)
    nblk = jnp.floor((cnt + (BLK - 1)) / BLK)

    eye8 = (lax.broadcasted_iota(jnp.int32, (E, E), 0)
            == lax.broadcasted_iota(jnp.int32, (E, E), 1)).astype(jnp.float32)
    nblk_col = jnp.sum(eye8 * nblk, axis=1, keepdims=True)
    l_strict = (lax.broadcasted_iota(jnp.int32, (E, E), 0)
                > lax.broadcasted_iota(jnp.int32, (E, E), 1)).astype(jnp.float32)
    l_incl = (lax.broadcasted_iota(jnp.int32, (E, E), 0)
              >= lax.broadcasted_iota(jnp.int32, (E, E), 1)).astype(jnp.float32)
    segstart_col = lax.dot_general(l_strict, nblk_col,
                                   (((1,), (0,)), ((), ()))) * BLK
    cumincl_col = lax.dot_general(l_incl, nblk_col,
                                  (((1,), (0,)), ((), ())))
    segstart_row = jnp.sum(eye8 * segstart_col, axis=0, keepdims=True)

    tril = (lax.broadcasted_iota(jnp.int32, (128, 128), 0)
            > lax.broadcasted_iota(jnp.int32, (128, 128), 1)).astype(jnp.float32)

    cnts = jnp.zeros((1, E), jnp.float32)
    rank_chunks = []
    for c in range((2 * T) // 128):
        blk = ofull[c * 128:(c + 1) * 128, :]
        pre = lax.dot_general(tril, blk, (((1,), (0,)), ((), ())))
        rank_chunks.append(pre + cnts)
        cnts = cnts + jnp.sum(blk, axis=0, keepdims=True)
    ranks = jnp.concatenate(rank_chunks, axis=0)

    pos = jnp.sum(ofull * (ranks + segstart_row), axis=-1, keepdims=True)
    pos = pos.astype(jnp.int32)
    p_ref[...] = jnp.concatenate([pos[:T], pos[T:]], axis=1)
    w_ref[...] = jnp.concatenate([w1, w2], axis=1)

    nba = jnp.sum(nblk_col)
    biota = lax.broadcasted_iota(jnp.int32, (1, 16), 1).astype(jnp.float32)
    cmp = (cumincl_col <= biota).astype(jnp.float32)
    gid = jnp.sum(cmp, axis=0, keepdims=True)
    gid_last = jnp.sum((cumincl_col <= (nba - 1.0)).astype(jnp.float32),
                       axis=0, keepdims=True)
    gid = jnp.where(biota < nba, gid, gid_last)
    lane = lax.broadcasted_iota(jnp.int32, (1, 16), 1)
    meta_ref[...] = jnp.where(lane == 15, nba.astype(jnp.int32),
                              gid.astype(jnp.int32))


def _router(xt, gate_w):
    return pl.pallas_call(
        _router_body,
        grid=(1,),
        in_specs=[pl.BlockSpec((T, H), lambda i: (0, 0)),
                  pl.BlockSpec((E, H), lambda i: (0, 0))],
        out_specs=[pl.BlockSpec((T, KTOP), lambda i: (0, 0)),
                   pl.BlockSpec((T, KTOP), lambda i: (0, 0)),
                   pl.BlockSpec((1, 16), lambda i: (0, 0))],
        out_shape=[jax.ShapeDtypeStruct((T, KTOP), jnp.int32),
                   jax.ShapeDtypeStruct((T, KTOP), jnp.float32),
                   jax.ShapeDtypeStruct((1, 16), jnp.int32)],
    )(xt, gate_w)


def _dispatch_sc(xt, pkm):
    mesh = plsc.VectorSubcoreMesh(core_axis_name="c", subcore_axis_name="s")

    @functools.partial(
        pl.kernel,
        out_type=jax.ShapeDtypeStruct((PADR, H), jnp.float32),
        mesh=mesh,
        scratch_types=[pltpu.VMEM((_NSUB, _SUB), jnp.int32),
                       pltpu.VMEM((_SUB, H), jnp.float32),
                       pltpu.SemaphoreType.DMA],
    )
    def body(x_hbm, pidx_hbm, xs_hbm, idx_v, rows_v, sem):
        wid = lax.axis_index("s") * 2 + lax.axis_index("c")
        pltpu.sync_copy(pidx_hbm.at[pl.ds(wid * _NSUB, _NSUB)], idx_v)
        tbase = lax.rem(wid * _SPW, T)
        for j in range(_NSUB):
            pltpu.sync_copy(x_hbm.at[pl.ds(tbase + j * _SUB, _SUB)], rows_v)
            pltpu.async_copy(rows_v, xs_hbm.at[idx_v.at[j]], sem).wait()

    return body(xt, pkm)


def _gather_sc(ys, pkm):
    mesh = plsc.VectorSubcoreMesh(core_axis_name="c", subcore_axis_name="s")

    @functools.partial(
        pl.kernel,
        out_type=jax.ShapeDtypeStruct((_SLOTS, H), jnp.float32),
        mesh=mesh,
        scratch_types=[pltpu.VMEM((_NSUB, _SUB), jnp.int32),
                       pltpu.VMEM((_SUB, H), jnp.float32),
                       pltpu.SemaphoreType.DMA],
    )
    def body(ys_hbm, pidx_hbm, g_hbm, idx_v, rows_v, sem):
        wid = lax.axis_index("s") * 2 + lax.axis_index("c")
        pltpu.sync_copy(pidx_hbm.at[pl.ds(wid * _NSUB, _NSUB)], idx_v)
        for j in range(_NSUB):
            pltpu.async_copy(ys_hbm.at[idx_v.at[j]], rows_v, sem).wait()
            pltpu.sync_copy(rows_v, g_hbm.at[pl.ds(wid * _SPW + j * _SUB, _SUB)])

    return body(ys, pkm)


def _ffn_grouped_body(meta_ref, xs_ref, wg_ref, wu_ref, wd_ref, out_ref):
    i = pl.program_id(0)
    j = pl.program_id(1)
    nba = meta_ref[15]

    @pl.when(i < nba)
    def _():
        xb = xs_ref[...]
        h1 = lax.dot_general(xb, wg_ref[0], (((1,), (1,)), ((), ())),
                             preferred_element_type=jnp.float32)
        h2 = lax.dot_general(xb, wu_ref[0], (((1,), (1,)), ((), ())),
                             preferred_element_type=jnp.float32)
        p = (h1 * lax.logistic(h1)) * h2
        part = lax.dot_general(p, wd_ref[0], (((1,), (0,)), ((), ())),
                               preferred_element_type=jnp.float32)

        @pl.when(j == 0)
        def _():
            out_ref[...] = part

        @pl.when(j > 0)
        def _():
            out_ref[...] += part


def _ffn_grouped(meta1, xs, Wg, Wu, Wd):
    def row_idx(i, j, m):
        return (jnp.minimum(i, m[15] - 1), 0)

    def jcol(i, j, m):
        return jnp.where(i < m[15], j, NIC - 1)

    grid_spec = pltpu.PrefetchScalarGridSpec(
        num_scalar_prefetch=1,
        grid=(NBR, NIC),
        in_specs=[
            pl.BlockSpec((BLK, H), row_idx),
            pl.BlockSpec((1, IC, H), lambda i, j, m: (m[i], jcol(i, j, m), 0)),
            pl.BlockSpec((1, IC, H), lambda i, j, m: (m[i], jcol(i, j, m), 0)),
            pl.BlockSpec((1, IC, H), lambda i, j, m: (m[i], jcol(i, j, m), 0)),
        ],
        out_specs=pl.BlockSpec((BLK, H), row_idx),
    )
    return pl.pallas_call(
        _ffn_grouped_body,
        grid_spec=grid_spec,
        out_shape=jax.ShapeDtypeStruct((PADR, H), jnp.float32),
        compiler_params=pltpu.CompilerParams(
            dimension_semantics=("arbitrary", "arbitrary")),
    )(meta1, xs, Wg, Wu, Wd)


def _ffn_shared_body(x_ref, sg_ref, su_ref, sd_ref, out_ref):
    j = pl.program_id(1)
    xb = x_ref[...]
    h1 = lax.dot_general(xb, sg_ref[...], (((1,), (1,)), ((), ())),
                         preferred_element_type=jnp.float32)
    h2 = lax.dot_general(xb, su_ref[...], (((1,), (1,)), ((), ())),
                         preferred_element_type=jnp.float32)
    p = (h1 * lax.logistic(h1)) * h2
    part = lax.dot_general(p, sd_ref[...], (((1,), (0,)), ((), ())),
                           preferred_element_type=jnp.float32)

    @pl.when(j == 0)
    def _():
        out_ref[...] = part

    @pl.when(j > 0)
    def _():
        out_ref[...] += part


def _ffn_shared(xt, Sg, Su, Sd):
    return pl.pallas_call(
        _ffn_shared_body,
        grid=(T // BLK, NIC),
        in_specs=[pl.BlockSpec((BLK, H), lambda i, j: (i, 0)),
                  pl.BlockSpec((IC, H), lambda i, j: (j, 0)),
                  pl.BlockSpec((IC, H), lambda i, j: (j, 0)),
                  pl.BlockSpec((IC, H), lambda i, j: (j, 0))],
        out_specs=pl.BlockSpec((BLK, H), lambda i, j: (i, 0)),
        out_shape=jax.ShapeDtypeStruct((T, H), jnp.float32),
        compiler_params=pltpu.CompilerParams(
            dimension_semantics=("arbitrary", "arbitrary")),
    )(xt, Sg, Su, Sd)


def _combine_body(g_ref, ysh_ref, w_ref, y_ref):
    g1 = g_ref[0]
    g2 = g_ref[1]
    w1 = w_ref[:, 0:1]
    w2 = w_ref[:, 1:2]
    y_ref[...] = w1 * g1 + w2 * g2 + ysh_ref[...]


def _combine(g, ysh, W):
    return pl.pallas_call(
        _combine_body,
        grid=(T // BLK,),
        in_specs=[pl.BlockSpec((KTOP, BLK, H), lambda i: (0, i, 0)),
                  pl.BlockSpec((BLK, H), lambda i: (i, 0)),
                  pl.BlockSpec((BLK, KTOP), lambda i: (i, 0))],
        out_specs=pl.BlockSpec((BLK, H), lambda i: (i, 0)),
        out_shape=jax.ShapeDtypeStruct((T, H), jnp.float32),
    )(g, ysh, W)


def kernel(x, gate_w, Wg, Wu, Wd, Sg, Su, Sd):
    b, s, h = x.shape
    xt = x.reshape(T, H)
    P, W, meta = _router(xt, gate_w)
    meta1 = meta.reshape(16)
    pkm = P.T.reshape(_SLOTS // _SUB, _SUB)
    xs = _dispatch_sc(xt, pkm)
    ysh = _ffn_shared(xt, Sg, Su, Sd.T)
    ys = _ffn_grouped(meta1, xs, Wg, Wu, Wd.transpose(0, 2, 1))
    g = _gather_sc(ys, pkm)
    y = _combine(g.reshape(KTOP, T, H), ysh, W)
    return y.reshape(b, s, h)
